# initial kernel scaffold (unmeasured)
import jax
import jax.numpy as jnp
from jax import lax
from jax.experimental import pallas as pl
from jax.experimental.pallas import tpu as pltpu


def kernel(
    x,
):
    def body(*refs):
        pass

    out_shape = jax.ShapeDtypeStruct(..., jnp.float32)
    return pl.pallas_call(body, out_shape=out_shape)(...)


# baseline (device time: 17497 ns/iter reference)
import jax
import jax.numpy as jnp
from jax import lax
from jax.experimental import pallas as pl
from jax.experimental.pallas import tpu as pltpu

N_DEV = 4


def _cmp_exchange(x, k_bit, d, flip):
    m = x.shape[0]
    idx = lax.broadcasted_iota(jnp.int32, x.shape, 0)
    is_lower = (idx & d) == 0
    asc = (idx & k_bit) == 0
    x_fwd = jnp.concatenate([x[d:], x[:d]], axis=0)
    x_bwd = jnp.concatenate([x[m - d:], x[:m - d]], axis=0)
    partner = jnp.where(is_lower, x_fwd, x_bwd)
    take_min = jnp.logical_not(jnp.logical_xor(asc, is_lower))
    if flip is not None:
        take_min = jnp.logical_xor(take_min, flip)
    return jnp.where(take_min, jnp.minimum(x, partner), jnp.maximum(x, partner))


def _bitonic_sort(x, flip):
    p = x.shape[0].bit_length() - 1
    for k in range(1, p + 1):
        for j in range(k - 1, -1, -1):
            x = _cmp_exchange(x, 1 << k, 1 << j, flip)
    return x


def _bitonic_stages(x, k_lo, k_hi):
    for k in range(k_lo, k_hi + 1):
        for j in range(k - 1, -1, -1):
            x = _cmp_exchange(x, 1 << k, 1 << j, None)
    return x


def kernel(x):
    m_per, n = x.shape
    m_global = N_DEV * m_per
    p_per = m_per.bit_length() - 1
    p_glob = m_global.bit_length() - 1

    def body(x_ref, out_ref, gather_ref, send_sems, recv_sems):
        my = lax.axis_index("i")
        right = lax.rem(my + 1, N_DEV)
        left = lax.rem(my + N_DEV - 1, N_DEV)

        barrier_sem = pltpu.get_barrier_semaphore()
        for nbr in (left, right):
            pl.semaphore_signal(
                barrier_sem, inc=1,
                device_id=(nbr,), device_id_type=pl.DeviceIdType.MESH,
            )
        pl.semaphore_wait(barrier_sem, 2)

        odd = lax.rem(my, 2) == 1
        gather_ref[my] = _bitonic_sort(x_ref[:, :], odd)

        for h in range(N_DEV - 1):
            origin = lax.rem(my - h + N_DEV, N_DEV)
            rdma = pltpu.make_async_remote_copy(
                src_ref=gather_ref.at[origin],
                dst_ref=gather_ref.at[origin],
                send_sem=send_sems.at[h],
                recv_sem=recv_sems.at[h],
                device_id=(right,),
                device_id_type=pl.DeviceIdType.MESH,
            )
            rdma.start()
            rdma.wait()

        full = gather_ref[:, :, :].reshape(m_global, n)
        merged = _bitonic_stages(full, p_per + 1, p_glob)
        gather_ref[:, :, :] = merged.reshape(N_DEV, m_per, n)
        out_ref[:, :] = gather_ref[my]

    return pl.pallas_call(
        body,
        out_shape=jax.ShapeDtypeStruct((m_per, n), x.dtype),
        in_specs=[pl.BlockSpec(memory_space=pltpu.VMEM)],
        out_specs=pl.BlockSpec(memory_space=pltpu.VMEM),
        scratch_shapes=[
            pltpu.VMEM((N_DEV, m_per, n), x.dtype),
            pltpu.SemaphoreType.DMA((N_DEV - 1,)),
            pltpu.SemaphoreType.DMA((N_DEV - 1,)),
        ],
        compiler_params=pltpu.CompilerParams(collective_id=0),
    )(x)


# device time: 14396 ns/iter; 1.2154x vs baseline; 1.2154x over previous
import jax
import jax.numpy as jnp
from jax import lax
from jax.experimental import pallas as pl
from jax.experimental.pallas import tpu as pltpu

N_DEV = 4


def _cmp_exchange(x, k_bit, d, flip):
    m = x.shape[0]
    idx = lax.broadcasted_iota(jnp.int32, x.shape, 0)
    is_lower = (idx & d) == 0
    asc = (idx & k_bit) == 0
    x_fwd = jnp.concatenate([x[d:], x[:d]], axis=0)
    x_bwd = jnp.concatenate([x[m - d:], x[:m - d]], axis=0)
    partner = jnp.where(is_lower, x_fwd, x_bwd)
    take_min = jnp.logical_not(jnp.logical_xor(asc, is_lower))
    if flip is not None:
        take_min = jnp.logical_xor(take_min, flip)
    return jnp.where(take_min, jnp.minimum(x, partner), jnp.maximum(x, partner))


def _bitonic_sort(x, flip):
    p = x.shape[0].bit_length() - 1
    for k in range(1, p + 1):
        for j in range(k - 1, -1, -1):
            x = _cmp_exchange(x, 1 << k, 1 << j, flip)
    return x


def _bitonic_merge(x, flip):
    m = x.shape[0]
    big = 4 * m
    for j in range((m.bit_length() - 1) - 1, -1, -1):
        x = _cmp_exchange(x, big, 1 << j, flip)
    return x


def kernel(x):
    m_per, n = x.shape

    def body(x_ref, out_ref, pair_ref, myblk_ref, otherblk_ref, sems):
        my = lax.axis_index("i")
        p1 = my ^ 1
        p3 = my ^ 3

        barrier_sem = pltpu.get_barrier_semaphore()
        for nbr in (p1, p3):
            pl.semaphore_signal(
                barrier_sem, inc=1,
                device_id=(nbr,), device_id_type=pl.DeviceIdType.MESH,
            )
        pl.semaphore_wait(barrier_sem, 2)

        odd = lax.rem(my, 2) == 1
        my_slot = my & 1
        pair_ref[my_slot] = _bitonic_sort(x_ref[:, :], odd)

        rdma_a = pltpu.make_async_remote_copy(
            src_ref=pair_ref.at[my_slot],
            dst_ref=pair_ref.at[my_slot],
            send_sem=sems.at[0],
            recv_sem=sems.at[1],
            device_id=(p1,),
            device_id_type=pl.DeviceIdType.MESH,
        )
        rdma_a.start()
        rdma_a.wait()

        flip9 = my >= 2
        pair = pair_ref[:, :, :].reshape(2 * m_per, n)
        myblk_ref[:, :] = _bitonic_merge(pair, flip9)

        rdma_b = pltpu.make_async_remote_copy(
            src_ref=myblk_ref,
            dst_ref=otherblk_ref,
            send_sem=sems.at[2],
            recv_sem=sems.at[3],
            device_id=(p3,),
            device_id_type=pl.DeviceIdType.MESH,
        )
        rdma_b.start()
        rdma_b.wait()

        lower = my < 2
        a = myblk_ref[:, :]
        b = otherblk_ref[:, :]
        c = jnp.where(lower, jnp.minimum(a, b), jnp.maximum(a, b))
        c = _bitonic_merge(c, None)
        myblk_ref[:, :] = c
        out_ref[:, :] = myblk_ref[pl.ds((my & 1) * m_per, m_per), :]

    return pl.pallas_call(
        body,
        out_shape=jax.ShapeDtypeStruct((m_per, n), x.dtype),
        in_specs=[pl.BlockSpec(memory_space=pltpu.VMEM)],
        out_specs=pl.BlockSpec(memory_space=pltpu.VMEM),
        scratch_shapes=[
            pltpu.VMEM((2, m_per, n), x.dtype),
            pltpu.VMEM((2 * m_per, n), x.dtype),
            pltpu.VMEM((2 * m_per, n), x.dtype),
            pltpu.SemaphoreType.DMA((4,)),
        ],
        compiler_params=pltpu.CompilerParams(collective_id=0),
    )(x)
